# Initial kernel scaffold; baseline (speedup 1.0000x reference)
import jax
import jax.numpy as jnp
from jax.experimental import pallas as pl


def kernel(x, mask):
    raise NotImplementedError("write your pallas kernel here")


# speedup vs baseline: 1.8522x; 1.8522x over previous
import functools

import jax
import jax.numpy as jnp
from jax import lax
from jax.experimental import pallas as pl
from jax.experimental.pallas import tpu as pltpu
from jax.experimental.pallas import tpu_sc as plsc

B = 4096
N = 8192
NC = 2
NS = 16
NW = NC * NS
ROWS_PER_W = B // NW
K = 4
GROUPS = ROWS_PER_W // K
LANES = 16
NVREG = N // LANES


def _masked_cumsum_body(x_hbm, m_hbm, out_hbm, xb, mb, ob):
    wid = lax.axis_index("s") * NC + lax.axis_index("c")
    base = wid * ROWS_PER_W

    def group(g, _):
        row0 = base + g * K
        pltpu.sync_copy(x_hbm.at[pl.ds(row0, K)], xb)
        pltpu.sync_copy(m_hbm.at[pl.ds(row0, K)], mb)

        def body(i, carries):
            col = pl.ds(i * LANES, LANES)
            new = []
            for k in range(K):
                xm = xb[k, col] * mb[k, col]
                s = plsc.cumsum(xm)
                ob[k, col] = s + carries[k]
                new.append(carries[k] + jnp.sum(xm))
            return tuple(new)

        lax.fori_loop(0, NVREG, body,
                      tuple(jnp.float32(0.0) for _ in range(K)))
        pltpu.sync_copy(ob, out_hbm.at[pl.ds(row0, K)])
        return 0

    lax.fori_loop(0, GROUPS, group, 0)


_mesh = plsc.VectorSubcoreMesh(core_axis_name="c", subcore_axis_name="s")

_masked_cumsum = functools.partial(
    pl.kernel,
    out_type=jax.ShapeDtypeStruct((B, N), jnp.float32),
    mesh=_mesh,
    compiler_params=pltpu.CompilerParams(needs_layout_passes=False),
    scratch_types=[
        pltpu.VMEM((K, N), jnp.float32),
        pltpu.VMEM((K, N), jnp.float32),
        pltpu.VMEM((K, N), jnp.float32),
    ],
)(_masked_cumsum_body)


def kernel(x, mask):
    return _masked_cumsum(x, mask.astype(jnp.float32))
